# baseline (speedup 1.0000x reference)
import functools

import jax
import jax.numpy as jnp
from jax import lax
from jax.experimental import pallas as pl
from jax.experimental.pallas import tpu as pltpu
from jax.experimental.pallas import tpu_sc as plsc

N = 10000
S = 2500
D = 128
KNN = 3
SP = 2560
ND = N - S
RB = 128
NDP = 7680
NBLK = NDP // RB
BIG = 1e30
FARPOS = 1e6
EPS_BN = 1e-5

NW = 32
RPW = NDP // NW
NCH = 6
CH = RPW // NCH
ECH = CH * KNN


def _mlp_kernel(feat1_ref, W1_ref, b1_ref, g1_ref, be1_ref,
                feat2_ref, W2_ref, b2_ref, g2_ref, be2_ref,
                h1_ref, h2_ref):
    pre1 = jnp.dot(feat1_ref[...], W1_ref[...],
                   preferred_element_type=jnp.float32) + b1_ref[...]
    mask1 = (jax.lax.broadcasted_iota(jnp.int32, (SP, 1), 0) < S
             ).astype(jnp.float32)
    m1 = jnp.sum(pre1 * mask1, axis=0, keepdims=True) * (1.0 / S)
    v1 = jnp.sum(((pre1 - m1) ** 2) * mask1, axis=0, keepdims=True) * (1.0 / S)
    y1 = (pre1 - m1) / jnp.sqrt(v1 + EPS_BN) * g1_ref[...] + be1_ref[...]
    h1_ref[...] = jnp.maximum(y1, 0.0) * mask1

    pre2 = jnp.dot(feat2_ref[...], W2_ref[...],
                   preferred_element_type=jnp.float32) + b2_ref[...]
    m2 = jnp.mean(pre2, axis=0, keepdims=True)
    v2 = jnp.mean((pre2 - m2) ** 2, axis=0, keepdims=True)
    y2 = (pre2 - m2) / jnp.sqrt(v2 + EPS_BN) * g2_ref[...] + be2_ref[...]
    h2_ref[...] = jnp.maximum(y2, 0.0)


def _knn_kernel(pd_ref, p1t_ref, p2t_ref, nbr_ref, wn_ref):
    pd = pd_ref[...]
    cross1 = jnp.dot(pd, p1t_ref[...], preferred_element_type=jnp.float32,
                     precision=jax.lax.Precision.HIGHEST)
    e2s = p1t_ref[3:4, :] - (cross1 + cross1)
    cross2 = jnp.dot(pd, p2t_ref[...], preferred_element_type=jnp.float32,
                     precision=jax.lax.Precision.HIGHEST)
    e2w = p2t_ref[3:4, :] - (cross2 + cross2)
    r2 = jnp.sum(pd * pd, axis=1, keepdims=True)
    lane = jax.lax.broadcasted_iota(jnp.int32, (RB, SP), 1)
    mds = []
    md_sum = jnp.zeros((RB, 1), jnp.float32)
    for k in range(KNN):
        mn = jnp.min(e2s, axis=1, keepdims=True)
        cand = jnp.where(e2s == mn, lane, SP)
        amin = jnp.min(cand, axis=1, keepdims=True)
        oh = lane == amin
        md = 1.0 / (jnp.sum(jnp.where(oh, e2w, 0.0), axis=1, keepdims=True)
                    + r2 + 1e-6)
        nbr_ref[:, k:k + 1] = amin
        mds.append(md)
        md_sum = md_sum + md
        e2s = jnp.where(oh, BIG, e2s)
    inv = 1.0 / md_sum
    wn_ref[...] = jnp.zeros((RB, 16), jnp.float32)
    for k in range(KNN):
        wn_ref[:, k:k + 1] = mds[k] * inv


def _sc_gather_kernel(h1_hbm, idx_hbm, w_hbm, h2d_hbm, out_hbm,
                      idx_v, rows_v, h2_v, w_v, sem):
    wid = lax.axis_index("s") * 2 + lax.axis_index("c")
    base = wid * RPW
    pltpu.sync_copy(idx_hbm.at[wid], idx_v)

    def start(j):
        return pltpu.async_copy(h1_hbm.at[idx_v.at[j]],
                                rows_v.at[j % 2], sem)

    gathers = [None] * NCH
    gathers[0] = start(0)
    gathers[1] = start(1)
    pltpu.sync_copy(h2d_hbm.at[pl.ds(base, RPW)], h2_v)
    pltpu.sync_copy(w_hbm.at[wid], w_v)
    for j in range(NCH):
        gathers[j].wait()
        buf = j % 2

        def body(r2, _):
            r = j * CH + r2
            wrow = w_v[r, :]
            w0, w1, w2 = wrow[0], wrow[1], wrow[2]
            for c in range(D // 16):
                sl = pl.ds(16 * c, 16)
                acc = h2_v[r, sl]
                acc = acc + w0 * rows_v[buf, 3 * r2, sl]
                acc = acc + w1 * rows_v[buf, 3 * r2 + 1, sl]
                acc = acc + w2 * rows_v[buf, 3 * r2 + 2, sl]
                h2_v[r, sl] = acc
            return 0

        lax.fori_loop(0, CH, body, 0)
        if j + 2 < NCH:
            gathers[j + 2] = start(j + 2)
    pltpu.sync_copy(h2_v, out_hbm.at[pl.ds(base, RPW)])


_sc_gather = functools.partial(
    pl.kernel,
    mesh=plsc.VectorSubcoreMesh(core_axis_name="c", subcore_axis_name="s"),
    out_type=jax.ShapeDtypeStruct((NDP, D), jnp.float32),
    scratch_types=[
        pltpu.VMEM((NCH, ECH), jnp.int32),
        pltpu.VMEM((2, ECH, D), jnp.float32),
        pltpu.VMEM((RPW, D), jnp.float32),
        pltpu.VMEM((RPW, 16), jnp.float32),
        pltpu.SemaphoreType.DMA,
    ],
)(_sc_gather_kernel)


@jax.jit
def _run(pos1, feat1, pos2, feat2, W1, b1, g1, be1, W2, b2, g2, be2):
    f32 = jnp.float32
    feat1p = jnp.zeros((SP, D), f32).at[:S].set(feat1)
    row = lambda v: v.reshape(1, D).astype(f32)
    h1p, h2 = pl.pallas_call(
        _mlp_kernel,
        out_shape=(jax.ShapeDtypeStruct((SP, D), f32),
                   jax.ShapeDtypeStruct((N, D), f32)),
    )(feat1p, W1.astype(f32), row(b1), row(g1), row(be1),
      feat2.astype(f32), W2.astype(f32), row(b2), row(g2), row(be2))

    p1t = jnp.full((8, SP), 0.0, f32).at[:3, :S].set(pos1.T)
    p1t = p1t.at[:3, S:].set(FARPOS)
    p1t = p1t.at[3, :].set(jnp.sum(p1t[:3, :] ** 2, axis=0))
    p2t = jnp.zeros((8, SP), f32).at[:3, :S].set(pos2[:S].T)
    p2t = p2t.at[3, :].set(jnp.sum(p2t[:3, :] ** 2, axis=0))
    pd = jnp.zeros((NDP, 8), f32).at[:ND, :3].set(pos2[S:])

    nbr, wn = pl.pallas_call(
        _knn_kernel,
        grid=(NBLK,),
        in_specs=[
            pl.BlockSpec((RB, 8), lambda i: (i, 0)),
            pl.BlockSpec((8, SP), lambda i: (0, 0)),
            pl.BlockSpec((8, SP), lambda i: (0, 0)),
        ],
        out_specs=(pl.BlockSpec((RB, KNN), lambda i: (i, 0)),
                   pl.BlockSpec((RB, 16), lambda i: (i, 0))),
        out_shape=(jax.ShapeDtypeStruct((NDP, KNN), jnp.int32),
                   jax.ShapeDtypeStruct((NDP, 16), f32)),
    )(pd, p1t, p2t)

    h2d = jnp.zeros((NDP, D), f32).at[:ND].set(h2[S:])
    idx3 = nbr.reshape(NW, NCH, ECH)
    w3 = wn.reshape(NW, RPW, 16)
    out_down = _sc_gather(h1p, idx3, w3, h2d)
    return jnp.concatenate([h2[:S], out_down[:ND]], axis=0)


def kernel(pos1, feat1, pos2, feat2, center, W1, b1, g1, be1, W2, b2, g2, be2):
    del center
    return _run(pos1, feat1, pos2, feat2, W1, b1, g1, be1, W2, b2, g2, be2)


# speedup vs baseline: 1.4185x; 1.4185x over previous
import functools

import jax
import jax.numpy as jnp
from jax import lax
from jax.experimental import pallas as pl
from jax.experimental.pallas import tpu as pltpu
from jax.experimental.pallas import tpu_sc as plsc

N = 10000
S = 2500
D = 128
KNN = 3
SP = 2560
ND = N - S
RB = 128
NDP = 7680
NBLK = NDP // RB
BIG = 1e30
FARPOS = 1e6
EPS_BN = 1e-5

NW = 32
RPW = NDP // NW
EPW = RPW * KNN
NCH = 6
CH = RPW // NCH
ECH = CH * KNN
NG = RPW // 16


def _mlp_kernel(feat1_ref, W1_ref, b1_ref, g1_ref, be1_ref,
                feat2_ref, W2_ref, b2_ref, g2_ref, be2_ref,
                h1_ref, h2_ref):
    pre1 = jnp.dot(feat1_ref[...], W1_ref[...],
                   preferred_element_type=jnp.float32) + b1_ref[...]
    mask1 = (jax.lax.broadcasted_iota(jnp.int32, (SP, 1), 0) < S
             ).astype(jnp.float32)
    m1 = jnp.sum(pre1 * mask1, axis=0, keepdims=True) * (1.0 / S)
    v1 = jnp.sum(((pre1 - m1) ** 2) * mask1, axis=0, keepdims=True) * (1.0 / S)
    y1 = (pre1 - m1) / jnp.sqrt(v1 + EPS_BN) * g1_ref[...] + be1_ref[...]
    h1_ref[...] = jnp.maximum(y1, 0.0) * mask1

    pre2 = jnp.dot(feat2_ref[...], W2_ref[...],
                   preferred_element_type=jnp.float32) + b2_ref[...]
    m2 = jnp.mean(pre2, axis=0, keepdims=True)
    v2 = jnp.mean((pre2 - m2) ** 2, axis=0, keepdims=True)
    y2 = (pre2 - m2) / jnp.sqrt(v2 + EPS_BN) * g2_ref[...] + be2_ref[...]
    h2_ref[...] = jnp.maximum(y2, 0.0)


def _knn_kernel(pd_ref, p1t_ref, nbr_ref):
    d2s = jnp.zeros((RB, SP), jnp.float32)
    for c in range(3):
        d2s = d2s + (pd_ref[:, c:c + 1] - p1t_ref[c:c + 1, :]) ** 2
    lane = jax.lax.broadcasted_iota(jnp.int32, (RB, SP), 1)
    for k in range(KNN):
        mn = jnp.min(d2s, axis=1, keepdims=True)
        cand = jnp.where(d2s == mn, lane, SP)
        amin = jnp.min(cand, axis=1, keepdims=True)
        nbr_ref[:, k:k + 1] = amin
        d2s = jnp.where(lane == amin, BIG, d2s)


def _sc_gather_kernel(h1_hbm, idx_hbm, idxT_hbm, p2c_hbm, pdc_hbm, h2d_hbm,
                      out_hbm,
                      idx_v, idxT_v, p2c_v, pdc_v, rows_v, h2_v, w_v, sem):
    wid = lax.axis_index("s") * 2 + lax.axis_index("c")
    base = wid * RPW
    pltpu.sync_copy(idx_hbm.at[wid], idx_v)

    def start(j):
        return pltpu.async_copy(h1_hbm.at[idx_v.at[j]],
                                rows_v.at[j % 2], sem)

    gathers = [None] * NCH
    gathers[0] = start(0)
    gathers[1] = start(1)
    pltpu.sync_copy(idxT_hbm.at[wid], idxT_v)
    pltpu.sync_copy(p2c_hbm, p2c_v)
    pltpu.sync_copy(pdc_hbm.at[wid], pdc_v)
    pltpu.sync_copy(h2d_hbm.at[pl.ds(base, RPW)], h2_v)

    zero16 = jnp.zeros((16,), jnp.int32)
    for g in range(NG):
        sl16 = pl.ds(16 * g, 16)
        dx = pdc_v[0, sl16]
        dy = pdc_v[1, sl16]
        dz = pdc_v[2, sl16]
        mds = []
        for k in range(KNN):
            src = idxT_v[k, sl16]
            gx = plsc.load_gather(p2c_v, [zero16, src])
            gy = plsc.load_gather(p2c_v, [zero16 + 1, src])
            gz = plsc.load_gather(p2c_v, [zero16 + 2, src])
            ex, ey, ez = gx - dx, gy - dy, gz - dz
            d2 = ex * ex + ey * ey + ez * ez
            mds.append(1.0 / (d2 + 1e-6))
        msum = mds[0] + mds[1] + mds[2]
        for k in range(KNN):
            w_v[k, sl16] = mds[k] / msum

    for j in range(NCH):
        gathers[j].wait()
        buf = j % 2

        def body(r2, _):
            r = j * CH + r2
            w0 = w_v[0, pl.ds(r, 16)][0]
            w1 = w_v[1, pl.ds(r, 16)][0]
            w2 = w_v[2, pl.ds(r, 16)][0]
            for c in range(D // 16):
                sl = pl.ds(16 * c, 16)
                acc = h2_v[r, sl]
                acc = acc + w0 * rows_v[buf, 3 * r2, sl]
                acc = acc + w1 * rows_v[buf, 3 * r2 + 1, sl]
                acc = acc + w2 * rows_v[buf, 3 * r2 + 2, sl]
                h2_v[r, sl] = acc
            return 0

        lax.fori_loop(0, CH, body, 0)
        if j + 2 < NCH:
            gathers[j + 2] = start(j + 2)
    pltpu.sync_copy(h2_v, out_hbm.at[pl.ds(base, RPW)])


_sc_gather = functools.partial(
    pl.kernel,
    mesh=plsc.VectorSubcoreMesh(core_axis_name="c", subcore_axis_name="s"),
    compiler_params=pltpu.CompilerParams(needs_layout_passes=False),
    out_type=jax.ShapeDtypeStruct((NDP, D), jnp.float32),
    scratch_types=[
        pltpu.VMEM((NCH, ECH), jnp.int32),
        pltpu.VMEM((KNN, RPW), jnp.int32),
        pltpu.VMEM((3, SP), jnp.float32),
        pltpu.VMEM((3, RPW), jnp.float32),
        pltpu.VMEM((2, ECH, D), jnp.float32),
        pltpu.VMEM((RPW, D), jnp.float32),
        pltpu.VMEM((KNN, RPW + 16), jnp.float32),
        pltpu.SemaphoreType.DMA,
    ],
)(_sc_gather_kernel)


@jax.jit
def _run(pos1, feat1, pos2, feat2, W1, b1, g1, be1, W2, b2, g2, be2):
    f32 = jnp.float32
    feat1p = jnp.zeros((SP, D), f32).at[:S].set(feat1)
    row = lambda v: v.reshape(1, D).astype(f32)
    h1p, h2 = pl.pallas_call(
        _mlp_kernel,
        out_shape=(jax.ShapeDtypeStruct((SP, D), f32),
                   jax.ShapeDtypeStruct((N, D), f32)),
    )(feat1p, W1.astype(f32), row(b1), row(g1), row(be1),
      feat2.astype(f32), W2.astype(f32), row(b2), row(g2), row(be2))

    p1t = jnp.full((8, SP), 0.0, f32).at[:3, :S].set(pos1.T)
    p1t = p1t.at[:3, S:].set(FARPOS)
    pd = jnp.zeros((NDP, 8), f32).at[:ND, :3].set(pos2[S:])

    nbr = pl.pallas_call(
        _knn_kernel,
        grid=(NBLK,),
        in_specs=[
            pl.BlockSpec((RB, 8), lambda i: (i, 0)),
            pl.BlockSpec((8, SP), lambda i: (0, 0)),
        ],
        out_specs=pl.BlockSpec((RB, KNN), lambda i: (i, 0)),
        out_shape=jax.ShapeDtypeStruct((NDP, KNN), jnp.int32),
    )(pd, p1t)

    h2d = jnp.zeros((NDP, D), f32).at[:ND].set(h2[S:])
    idx3 = nbr.reshape(NW, NCH, ECH)
    idxT = nbr.reshape(NW, RPW, KNN).transpose(0, 2, 1)
    p2c = jnp.zeros((3, SP), f32).at[:, :S].set(pos2[:S].T)
    pdc = pd[:, :3].T.reshape(3, NW, RPW).transpose(1, 0, 2)
    out_down = _sc_gather(h1p, idx3, idxT, p2c, pdc, h2d)
    return jnp.concatenate([h2[:S], out_down[:ND]], axis=0)


def kernel(pos1, feat1, pos2, feat2, center, W1, b1, g1, be1, W2, b2, g2, be2):
    del center
    return _run(pos1, feat1, pos2, feat2, W1, b1, g1, be1, W2, b2, g2, be2)
